# baseline (speedup 1.0000x reference)
import functools

import jax
import jax.numpy as jnp
from jax import lax
from jax.experimental import pallas as pl
from jax.experimental.pallas import tpu as pltpu
from jax.experimental.pallas import tpu_sc as plsc

N_NODES = 10000
N_EDGES = 320000
D_FEAT = 128
D_EDGE = 16
D_OUT = 16
N_PK = N_EDGES // 8


def _nodeproj_body(x_ref, w_ref, xs_ref, xd_ref):
    xs_ref[...] = jnp.dot(x_ref[...], w_ref[D_EDGE:D_EDGE + D_FEAT, :],
                          preferred_element_type=jnp.float32)
    xd_ref[...] = jnp.dot(x_ref[...], w_ref[D_EDGE + D_FEAT:, :],
                          preferred_element_type=jnp.float32)


def _nodeproj(x, W):
    return pl.pallas_call(
        _nodeproj_body,
        out_shape=[
            jax.ShapeDtypeStruct((N_NODES, D_OUT), jnp.float32),
            jax.ShapeDtypeStruct((N_NODES, D_OUT), jnp.float32),
        ],
    )(x, W)


_EBLK = 8000
_NBLK = N_EDGES // _EBLK
_PBLK = _EBLK // 8


def _eproj_body(ea_ref, w_ref, b_ref, out_ref):
    t = (jnp.dot(ea_ref[...], w_ref[:D_EDGE, :],
                 preferred_element_type=jnp.float32) + b_ref[...])
    out_ref[...] = jnp.concatenate(
        [t[j * _PBLK:(j + 1) * _PBLK, :] for j in range(8)], axis=1)


def _eproj(edge_attr, W, b2d):
    return pl.pallas_call(
        _eproj_body,
        grid=(_NBLK,),
        in_specs=[
            pl.BlockSpec((_EBLK, D_EDGE), lambda i: (i, 0)),
            pl.BlockSpec((D_EDGE + 2 * D_FEAT, D_OUT), lambda i: (0, 0)),
            pl.BlockSpec((1, D_OUT), lambda i: (0, 0)),
        ],
        out_specs=pl.BlockSpec((_PBLK, 128), lambda i: (i, 0)),
        out_shape=jax.ShapeDtypeStruct((N_PK, 128), jnp.float32),
    )(edge_attr, W, b2d)


def _unpack_body(in_ref, out_ref):
    p = in_ref[...]
    out_ref[...] = jnp.concatenate(
        [p[:, j * D_OUT:(j + 1) * D_OUT] for j in range(8)], axis=0)


def _unpack(packed):
    return pl.pallas_call(
        _unpack_body,
        grid=(_NBLK,),
        in_specs=[pl.BlockSpec((_PBLK, 128), lambda i: (i, 0))],
        out_specs=pl.BlockSpec((_EBLK, D_OUT), lambda i: (i, 0)),
        out_shape=jax.ShapeDtypeStruct((N_EDGES, D_OUT), jnp.float32),
    )(packed)


_NW = 32
_EPW = N_EDGES // _NW
_MACRO = 2000
_PMACRO = _MACRO // 8
_NMACRO = _EPW // _MACRO

_mesh = plsc.VectorSubcoreMesh(core_axis_name="c", subcore_axis_name="s")


@functools.partial(
    pl.kernel,
    mesh=_mesh,
    compiler_params=pltpu.CompilerParams(use_tc_tiling_on_sc=False),
    out_type=jax.ShapeDtypeStruct((N_PK * 128,), jnp.float32),
    scratch_types=[
        pltpu.VMEM((_EPW,), jnp.int32),
        pltpu.VMEM((_EPW,), jnp.int32),
        pltpu.VMEM((_MACRO, D_OUT), jnp.float32),
        pltpu.VMEM((_MACRO, D_OUT), jnp.float32),
        pltpu.VMEM((_MACRO * D_OUT,), jnp.float32),
        pltpu.SemaphoreType.DMA,
        pltpu.SemaphoreType.DMA,
        pltpu.SemaphoreType.DMA,
    ],
)
def _sc_gather_add(src_hbm, dst_hbm, xs_hbm, xd_hbm, ea_hbm, out_hbm,
                   idx_s, idx_d, rows_s, rows_d, acc, sem_s, sem_d, sem_e):
    wid = lax.axis_index("s") * 2 + lax.axis_index("c")
    base = wid * _EPW
    pltpu.sync_copy(src_hbm.at[pl.ds(base, _EPW)], idx_s)
    pltpu.sync_copy(dst_hbm.at[pl.ds(base, _EPW)], idx_d)
    for m in range(_NMACRO):
        off = m * _MACRO
        foff = (base + off) * D_OUT
        cp_e = pltpu.async_copy(ea_hbm.at[pl.ds(foff, _MACRO * D_OUT)], acc, sem_e)
        cp_s = pltpu.async_copy(xs_hbm.at[idx_s.at[pl.ds(off, _MACRO)]], rows_s, sem_s)
        cp_d = pltpu.async_copy(xd_hbm.at[idx_d.at[pl.ds(off, _MACRO)]], rows_d, sem_d)
        cp_e.wait()
        cp_s.wait()
        cp_d.wait()

        def body(f, _):
            sl = pl.ds(f * D_OUT, D_OUT)
            acc[sl] = acc[sl] + rows_s[f, :] + rows_d[f, :]
            return 0

        lax.fori_loop(0, _MACRO, body, 0)
        pltpu.sync_copy(acc, out_hbm.at[pl.ds(foff, _MACRO * D_OUT)])


def _to_packed_order(v):
    return v.reshape(_NBLK, 8, _PBLK).transpose(0, 2, 1).reshape(N_EDGES)


def kernel(x, edge_index, edge_attr, pos, W, b):
    src = _to_packed_order(edge_index[0])
    dst = _to_packed_order(edge_index[1])
    xs, xd = _nodeproj(x, W)
    eap = _eproj(edge_attr, W, b.reshape(1, D_OUT))
    out_flat = _sc_gather_add(src, dst, xs, xd, eap.reshape(-1))
    return (x, _unpack(out_flat.reshape(N_PK, 128)), edge_index, pos)


# speedup vs baseline: 1.7893x; 1.7893x over previous
import functools

import jax
import jax.numpy as jnp
from jax import lax
from jax.experimental import pallas as pl
from jax.experimental.pallas import tpu as pltpu
from jax.experimental.pallas import tpu_sc as plsc

N_NODES = 10000
N_EDGES = 320000
D_FEAT = 128
D_EDGE = 16
D_OUT = 16
N_PK = N_EDGES // 8


def _nodeproj_body(x_ref, w_ref, xs_ref, xd_ref):
    xs_ref[...] = jnp.dot(x_ref[...], w_ref[D_EDGE:D_EDGE + D_FEAT, :],
                          preferred_element_type=jnp.float32)
    xd_ref[...] = jnp.dot(x_ref[...], w_ref[D_EDGE + D_FEAT:, :],
                          preferred_element_type=jnp.float32)


def _nodeproj(x, W):
    return pl.pallas_call(
        _nodeproj_body,
        out_shape=[
            jax.ShapeDtypeStruct((N_NODES, D_OUT), jnp.float32),
            jax.ShapeDtypeStruct((N_NODES, D_OUT), jnp.float32),
        ],
    )(x, W)


_G = 2000
_EBLK = 8 * _G
_NBLK = N_EDGES // _EBLK

_CONTRACT0 = (((0,), (0,)), ((), ()))
_CONTRACT1 = (((1,), (1,)), ((), ()))


def _eproj_body(eat_ref, w_ref, b_ref, out_ref):
    eat = eat_ref[...]
    we = w_ref[:D_EDGE, :]
    parts = []
    for j in range(8):
        sl = eat[:, j * _G:(j + 1) * _G]
        parts.append(lax.dot_general(sl, we, _CONTRACT0,
                                     preferred_element_type=jnp.float32))
    out_ref[...] = jnp.concatenate(parts, axis=1) + b_ref[...]


def _eproj(eaT, W, b128):
    return pl.pallas_call(
        _eproj_body,
        grid=(_NBLK,),
        in_specs=[
            pl.BlockSpec((D_EDGE, _EBLK), lambda i: (0, i)),
            pl.BlockSpec((D_EDGE + 2 * D_FEAT, D_OUT), lambda i: (0, 0)),
            pl.BlockSpec((1, 128), lambda i: (0, 0)),
        ],
        out_specs=pl.BlockSpec((_G, 128), lambda i: (i, 0)),
        out_shape=jax.ShapeDtypeStruct((N_PK, 128), jnp.float32),
    )(eaT, W, b128)


def _unpack_body(in_ref, eye_ref, out_ref):
    p = in_ref[...]
    eye = eye_ref[...]
    parts = []
    for j in range(8):
        sl = p[:, j * D_OUT:(j + 1) * D_OUT]
        parts.append(lax.dot_general(eye, sl, _CONTRACT1,
                                     preferred_element_type=jnp.float32))
    out_ref[...] = jnp.concatenate(parts, axis=1)


def _unpack(packed, eye16):
    return pl.pallas_call(
        _unpack_body,
        grid=(_NBLK,),
        in_specs=[
            pl.BlockSpec((_G, 128), lambda i: (i, 0)),
            pl.BlockSpec((D_OUT, D_OUT), lambda i: (0, 0)),
        ],
        out_specs=pl.BlockSpec((D_EDGE, _EBLK), lambda i: (0, i)),
        out_shape=jax.ShapeDtypeStruct((D_OUT, N_EDGES), jnp.float32),
    )(packed, eye16)


_NW = 32
_EPW = N_EDGES // _NW
_MACRO = 2000
_PMACRO = _MACRO // 8
_NMACRO = _EPW // _MACRO

_mesh = plsc.VectorSubcoreMesh(core_axis_name="c", subcore_axis_name="s")


@functools.partial(
    pl.kernel,
    mesh=_mesh,
    compiler_params=pltpu.CompilerParams(use_tc_tiling_on_sc=False),
    out_type=jax.ShapeDtypeStruct((N_PK, 128), jnp.float32),
    scratch_types=[
        pltpu.VMEM((_EPW,), jnp.int32),
        pltpu.VMEM((_EPW,), jnp.int32),
        pltpu.VMEM((_MACRO, D_OUT), jnp.float32),
        pltpu.VMEM((_MACRO, D_OUT), jnp.float32),
        pltpu.VMEM((_PMACRO, 128), jnp.float32),
        pltpu.SemaphoreType.DMA,
        pltpu.SemaphoreType.DMA,
        pltpu.SemaphoreType.DMA,
    ],
)
def _sc_gather_add(src_hbm, dst_hbm, xs_hbm, xd_hbm, ea_hbm, out_hbm,
                   idx_s, idx_d, rows_s, rows_d, acc, sem_s, sem_d, sem_e):
    wid = lax.axis_index("s") * 2 + lax.axis_index("c")
    base = wid * _EPW
    pltpu.sync_copy(src_hbm.at[pl.ds(base, _EPW)], idx_s)
    pltpu.sync_copy(dst_hbm.at[pl.ds(base, _EPW)], idx_d)
    for m in range(_NMACRO):
        off = m * _MACRO
        poff = (base + off) // 8
        cp_e = pltpu.async_copy(ea_hbm.at[pl.ds(poff, _PMACRO), :], acc, sem_e)
        cp_s = pltpu.async_copy(xs_hbm.at[idx_s.at[pl.ds(off, _MACRO)]], rows_s, sem_s)
        cp_d = pltpu.async_copy(xd_hbm.at[idx_d.at[pl.ds(off, _MACRO)]], rows_d, sem_d)
        cp_e.wait()
        cp_s.wait()
        cp_d.wait()

        def body(r2, _):
            for k in range(8):
                sl = pl.ds(k * D_OUT, D_OUT)
                acc[r2, sl] = (acc[r2, sl]
                               + rows_s[r2 * 8 + k, :] + rows_d[r2 * 8 + k, :])
            return 0

        lax.fori_loop(0, _PMACRO, body, 0)
        pltpu.sync_copy(acc, out_hbm.at[pl.ds(poff, _PMACRO), :])


def _to_packed_order(v):
    return v.reshape(_NBLK, 8, _G).transpose(0, 2, 1).reshape(N_EDGES)


def kernel(x, edge_index, edge_attr, pos, W, b):
    src = _to_packed_order(edge_index[0])
    dst = _to_packed_order(edge_index[1])
    xs, xd = _nodeproj(x, W)
    eap = _eproj(edge_attr.T, W, jnp.tile(b, 8).reshape(1, 128))
    out_pk = _sc_gather_add(src, dst, xs, xd, eap)
    outT = _unpack(out_pk, jnp.eye(D_OUT, dtype=jnp.float32))
    return (x, outT.T, edge_index, pos)


# speedup vs baseline: 2.9155x; 1.6294x over previous
import functools

import jax
import jax.numpy as jnp
from jax import lax
from jax.experimental import pallas as pl
from jax.experimental.pallas import tpu as pltpu
from jax.experimental.pallas import tpu_sc as plsc

N_NODES = 10000
N_EDGES = 320000
D_FEAT = 128
D_EDGE = 16
D_OUT = 16
N_PK = N_EDGES // 8


def _nodeproj_body(x_ref, w_ref, xs_ref, xd_ref):
    xs_ref[...] = jnp.dot(x_ref[...], w_ref[D_EDGE:D_EDGE + D_FEAT, :],
                          preferred_element_type=jnp.float32)
    xd_ref[...] = jnp.dot(x_ref[...], w_ref[D_EDGE + D_FEAT:, :],
                          preferred_element_type=jnp.float32)


def _nodeproj(x, W):
    return pl.pallas_call(
        _nodeproj_body,
        out_shape=[
            jax.ShapeDtypeStruct((N_NODES, D_OUT), jnp.float32),
            jax.ShapeDtypeStruct((N_NODES, D_OUT), jnp.float32),
        ],
    )(x, W)


_G = 2000
_EBLK = 8 * _G
_NBLK = N_EDGES // _EBLK

_CONTRACT0 = (((0,), (0,)), ((), ()))
_CONTRACT1 = (((1,), (1,)), ((), ()))


def _finish_body(gs_ref, eat_ref, w_ref, b_ref, eye_ref, out_ref):
    p = gs_ref[...]
    eye = eye_ref[...]
    parts = []
    for j in range(8):
        sl = p[:, j * D_OUT:(j + 1) * D_OUT]
        parts.append(lax.dot_general(eye, sl, _CONTRACT1,
                                     preferred_element_type=jnp.float32))
    gsT = jnp.concatenate(parts, axis=1)
    eapT = lax.dot_general(w_ref[:D_EDGE, :], eat_ref[...], _CONTRACT0,
                           preferred_element_type=jnp.float32)
    out_ref[...] = gsT + eapT + b_ref[...]


def _finish(gsum_pk, eaT, W, bcol, eye16):
    return pl.pallas_call(
        _finish_body,
        grid=(_NBLK,),
        in_specs=[
            pl.BlockSpec((_G, 128), lambda i: (i, 0)),
            pl.BlockSpec((D_EDGE, _EBLK), lambda i: (0, i)),
            pl.BlockSpec((D_EDGE + 2 * D_FEAT, D_OUT), lambda i: (0, 0)),
            pl.BlockSpec((D_OUT, 1), lambda i: (0, 0)),
            pl.BlockSpec((D_OUT, D_OUT), lambda i: (0, 0)),
        ],
        out_specs=pl.BlockSpec((D_OUT, _EBLK), lambda i: (0, i)),
        out_shape=jax.ShapeDtypeStruct((D_OUT, N_EDGES), jnp.float32),
    )(gsum_pk, eaT, W, bcol, eye16)


_NW = 32
_EPW = N_EDGES // _NW
_MACRO = 2000
_PMACRO = _MACRO // 8
_NMACRO = _EPW // _MACRO

_mesh = plsc.VectorSubcoreMesh(core_axis_name="c", subcore_axis_name="s")


@functools.partial(
    pl.kernel,
    mesh=_mesh,
    compiler_params=pltpu.CompilerParams(use_tc_tiling_on_sc=False),
    out_type=jax.ShapeDtypeStruct((N_PK, 128), jnp.float32),
    scratch_types=[
        pltpu.VMEM((2 * _EPW,), jnp.int32),
        pltpu.VMEM((_MACRO, D_OUT), jnp.float32),
        pltpu.VMEM((_MACRO, D_OUT), jnp.float32),
        pltpu.VMEM((_PMACRO, 128), jnp.float32),
        pltpu.SemaphoreType.DMA,
        pltpu.SemaphoreType.DMA,
    ],
)
def _sc_gather_sum(sd_hbm, xs_hbm, xd_hbm, out_hbm,
                   idx, rows_s, rows_d, acc, sem_s, sem_d):
    wid = lax.axis_index("s") * 2 + lax.axis_index("c")
    base = wid * _EPW
    pltpu.sync_copy(sd_hbm.at[pl.ds(base, _EPW)], idx.at[pl.ds(0, _EPW)])
    pltpu.sync_copy(sd_hbm.at[pl.ds(N_EDGES + base, _EPW)],
                    idx.at[pl.ds(_EPW, _EPW)])
    for m in range(_NMACRO):
        off = m * _MACRO
        poff = (base + off) // 8
        cp_s = pltpu.async_copy(xs_hbm.at[idx.at[pl.ds(off, _MACRO)]],
                                rows_s, sem_s)
        cp_d = pltpu.async_copy(xd_hbm.at[idx.at[pl.ds(_EPW + off, _MACRO)]],
                                rows_d, sem_d)
        cp_s.wait()
        cp_d.wait()

        def body(r2, _):
            for k in range(8):
                sl = pl.ds(k * D_OUT, D_OUT)
                acc[r2, sl] = rows_s[r2 * 8 + k, :] + rows_d[r2 * 8 + k, :]
            return 0

        lax.fori_loop(0, _PMACRO, body, 0)
        pltpu.sync_copy(acc, out_hbm.at[pl.ds(poff, _PMACRO), :])


def kernel(x, edge_index, edge_attr, pos, W, b):
    sd = (edge_index.reshape(2, _NBLK, 8, _G)
          .transpose(0, 1, 3, 2).reshape(2 * N_EDGES))
    xs, xd = _nodeproj(x, W)
    gsum_pk = _sc_gather_sum(sd, xs, xd)
    outT = _finish(gsum_pk, edge_attr.T, W, b.reshape(D_OUT, 1),
                   jnp.eye(D_OUT, dtype=jnp.float32))
    return (x, outT.T, edge_index, pos)


# speedup vs baseline: 2.9708x; 1.0190x over previous
import functools

import jax
import jax.numpy as jnp
from jax import lax
from jax.experimental import pallas as pl
from jax.experimental.pallas import tpu as pltpu
from jax.experimental.pallas import tpu_sc as plsc

N_NODES = 10000
N_EDGES = 320000
D_FEAT = 128
D_EDGE = 16
D_OUT = 16
N_PK = N_EDGES // 8


def _nodeproj_body(x_ref, w_ref, xs_ref, xd_ref):
    xs_ref[...] = jnp.dot(x_ref[...], w_ref[D_EDGE:D_EDGE + D_FEAT, :],
                          preferred_element_type=jnp.float32)
    xd_ref[...] = jnp.dot(x_ref[...], w_ref[D_EDGE + D_FEAT:, :],
                          preferred_element_type=jnp.float32)


def _nodeproj(x, W):
    return pl.pallas_call(
        _nodeproj_body,
        out_shape=[
            jax.ShapeDtypeStruct((N_NODES, D_OUT), jnp.float32),
            jax.ShapeDtypeStruct((N_NODES, D_OUT), jnp.float32),
        ],
    )(x, W)


_G = 2000
_EBLK = 8 * _G
_NBLK = N_EDGES // _EBLK

_CONTRACT0 = (((0,), (0,)), ((), ()))
_CONTRACT1 = (((1,), (1,)), ((), ()))


def _finish_body(gs_ref, eat_ref, w_ref, b_ref, eye_ref, out_ref):
    p = gs_ref[...]
    eye = eye_ref[...]
    parts = []
    for j in range(8):
        sl = p[:, j * D_OUT:(j + 1) * D_OUT]
        parts.append(lax.dot_general(eye, sl, _CONTRACT1,
                                     preferred_element_type=jnp.float32))
    gsT = jnp.concatenate(parts, axis=1)
    eapT = lax.dot_general(w_ref[:D_EDGE, :], eat_ref[...], _CONTRACT0,
                           preferred_element_type=jnp.float32)
    out_ref[...] = gsT + eapT + b_ref[...]


def _finish(gsum_pk, eaT, W, bcol, eye16):
    return pl.pallas_call(
        _finish_body,
        grid=(_NBLK,),
        in_specs=[
            pl.BlockSpec((_G, 128), lambda i: (i, 0)),
            pl.BlockSpec((D_EDGE, _EBLK), lambda i: (0, i)),
            pl.BlockSpec((D_EDGE + 2 * D_FEAT, D_OUT), lambda i: (0, 0)),
            pl.BlockSpec((D_OUT, 1), lambda i: (0, 0)),
            pl.BlockSpec((D_OUT, D_OUT), lambda i: (0, 0)),
        ],
        out_specs=pl.BlockSpec((D_OUT, _EBLK), lambda i: (0, i)),
        out_shape=jax.ShapeDtypeStruct((D_OUT, N_EDGES), jnp.float32),
    )(gsum_pk, eaT, W, bcol, eye16)


_NW = 32
_EPW = N_EDGES // _NW
_MACRO = 2000
_PMACRO = _MACRO // 8
_NMACRO = _EPW // _MACRO

_mesh = plsc.VectorSubcoreMesh(core_axis_name="c", subcore_axis_name="s")


@functools.partial(
    pl.kernel,
    mesh=_mesh,
    compiler_params=pltpu.CompilerParams(use_tc_tiling_on_sc=False),
    out_type=jax.ShapeDtypeStruct((N_PK, 128), jnp.float32),
    scratch_types=[
        pltpu.VMEM((_EPW,), jnp.int32),
        pltpu.VMEM((_EPW,), jnp.int32),
        pltpu.VMEM((_MACRO, D_OUT), jnp.float32),
        pltpu.VMEM((_MACRO, D_OUT), jnp.float32),
        pltpu.VMEM((_MACRO, D_OUT), jnp.float32),
        pltpu.SemaphoreType.DMA,
        pltpu.SemaphoreType.DMA,
    ],
)
def _sc_gather_sum(src_hbm, dst_hbm, xs_hbm, xd_hbm, out_hbm,
                   idx_s, idx_d, rows_s, rows_d, res, sem_s, sem_d):
    wid = lax.axis_index("s") * 2 + lax.axis_index("c")
    base = wid * _EPW
    pltpu.sync_copy(src_hbm.at[pl.ds(base, _EPW)], idx_s)
    pltpu.sync_copy(dst_hbm.at[pl.ds(base, _EPW)], idx_d)
    for m in range(_NMACRO):
        off = m * _MACRO
        gm = (base + off) // _MACRO
        prow = (gm // 8) * _G
        j = gm % 8
        cp_s = pltpu.async_copy(xs_hbm.at[idx_s.at[pl.ds(off, _MACRO)]],
                                rows_s, sem_s)
        cp_d = pltpu.async_copy(xd_hbm.at[idx_d.at[pl.ds(off, _MACRO)]],
                                rows_d, sem_d)
        cp_s.wait()
        cp_d.wait()

        def body(r, _):
            res[r, :] = rows_s[r, :] + rows_d[r, :]
            return 0

        lax.fori_loop(0, _MACRO, body, 0)
        pltpu.sync_copy(res, out_hbm.at[pl.ds(prow, _G),
                                        pl.ds(j * D_OUT, D_OUT)])


def kernel(x, edge_index, edge_attr, pos, W, b):
    xs, xd = _nodeproj(x, W)
    gsum_pk = _sc_gather_sum(edge_index[0], edge_index[1], xs, xd)
    outT = _finish(gsum_pk, edge_attr.T, W, b.reshape(D_OUT, 1),
                   jnp.eye(D_OUT, dtype=jnp.float32))
    return (x, outT.T, edge_index, pos)


# speedup vs baseline: 3.7422x; 1.2597x over previous
import functools

import jax
import jax.numpy as jnp
from jax import lax
from jax.experimental import pallas as pl
from jax.experimental.pallas import tpu as pltpu
from jax.experimental.pallas import tpu_sc as plsc

N_NODES = 10000
N_EDGES = 320000
D_FEAT = 128
D_EDGE = 16
D_OUT = 16
N_PK = N_EDGES // 8


def _nodeproj_body(x_ref, w_ref, xs_ref, xd_ref):
    xs_ref[...] = jnp.dot(x_ref[...], w_ref[D_EDGE:D_EDGE + D_FEAT, :],
                          preferred_element_type=jnp.float32)
    xd_ref[...] = jnp.dot(x_ref[...], w_ref[D_EDGE + D_FEAT:, :],
                          preferred_element_type=jnp.float32)


def _nodeproj(x, W):
    return pl.pallas_call(
        _nodeproj_body,
        out_shape=[
            jax.ShapeDtypeStruct((N_NODES, D_OUT), jnp.float32),
            jax.ShapeDtypeStruct((N_NODES, D_OUT), jnp.float32),
        ],
    )(x, W)


_G = 2000
_EBLK = 8 * _G
_NBLK = N_EDGES // _EBLK

_CONTRACT0 = (((0,), (0,)), ((), ()))
_CONTRACT1 = (((1,), (1,)), ((), ()))


def _finish_body(gs_ref, eat_ref, w_ref, b_ref, eye_ref, out_ref):
    p = gs_ref[...]
    eye = eye_ref[...]
    parts = []
    for j in range(8):
        sl = p[:, j * D_OUT:(j + 1) * D_OUT]
        parts.append(lax.dot_general(eye, sl, _CONTRACT1,
                                     preferred_element_type=jnp.float32))
    gsT = jnp.concatenate(parts, axis=1)
    eapT = lax.dot_general(w_ref[:D_EDGE, :], eat_ref[...], _CONTRACT0,
                           preferred_element_type=jnp.float32)
    out_ref[...] = gsT + eapT + b_ref[...]


def _finish(gsum_pk, eaT, W, bcol, eye16):
    return pl.pallas_call(
        _finish_body,
        grid=(_NBLK,),
        in_specs=[
            pl.BlockSpec((_G, 128), lambda i: (i, 0)),
            pl.BlockSpec((D_EDGE, _EBLK), lambda i: (0, i)),
            pl.BlockSpec((D_EDGE + 2 * D_FEAT, D_OUT), lambda i: (0, 0)),
            pl.BlockSpec((D_OUT, 1), lambda i: (0, 0)),
            pl.BlockSpec((D_OUT, D_OUT), lambda i: (0, 0)),
        ],
        out_specs=pl.BlockSpec((D_OUT, _EBLK), lambda i: (0, i)),
        out_shape=jax.ShapeDtypeStruct((D_OUT, N_EDGES), jnp.float32),
    )(gsum_pk, eaT, W, bcol, eye16)


_NW = 32
_EPW = N_EDGES // _NW
_MACRO = 2000
_PMACRO = _MACRO // 8
_NMACRO = _EPW // _MACRO

_mesh = plsc.VectorSubcoreMesh(core_axis_name="c", subcore_axis_name="s")


@functools.partial(
    pl.kernel,
    mesh=_mesh,
    compiler_params=pltpu.CompilerParams(use_tc_tiling_on_sc=False),
    out_type=jax.ShapeDtypeStruct((N_PK, 128), jnp.float32),
    scratch_types=[
        pltpu.VMEM((_EPW,), jnp.int32),
        pltpu.VMEM((_EPW,), jnp.int32),
        pltpu.VMEM((_MACRO, D_OUT), jnp.float32),
        pltpu.VMEM((_MACRO, D_OUT), jnp.float32),
        pltpu.VMEM((_MACRO, D_OUT), jnp.float32),
        pltpu.SemaphoreType.DMA,
        pltpu.SemaphoreType.DMA,
    ],
)
def _sc_gather_sum(src_hbm, dst_hbm, xs_hbm, xd_hbm, out_hbm,
                   idx_s, idx_d, rows_s, rows_d, res, sem_s, sem_d):
    wid = lax.axis_index("s") * 2 + lax.axis_index("c")
    base = wid * _EPW
    pltpu.sync_copy(src_hbm.at[pl.ds(base, _EPW)], idx_s)
    pltpu.sync_copy(dst_hbm.at[pl.ds(base, _EPW)], idx_d)
    for m in range(_NMACRO):
        off = m * _MACRO
        gm = (base + off) // _MACRO
        prow = (gm // 8) * _G
        j = gm % 8
        cp_s = pltpu.async_copy(xs_hbm.at[idx_s.at[pl.ds(off, _MACRO)]],
                                res, sem_s)
        cp_s.wait()
        cp_d = pltpu.async_copy(xd_hbm.at[idx_d.at[pl.ds(off, _MACRO)]],
                                res, sem_d, add=True)
        cp_d.wait()
        pltpu.sync_copy(res, out_hbm.at[pl.ds(prow, _G),
                                        pl.ds(j * D_OUT, D_OUT)])


def kernel(x, edge_index, edge_attr, pos, W, b):
    xs, xd = _nodeproj(x, W)
    gsum_pk = _sc_gather_sum(edge_index[0], edge_index[1], xs, xd)
    outT = _finish(gsum_pk, edge_attr.T, W, b.reshape(D_OUT, 1),
                   jnp.eye(D_OUT, dtype=jnp.float32))
    return (x, outT.T, edge_index, pos)


# speedup vs baseline: 3.7985x; 1.0150x over previous
import functools

import jax
import jax.numpy as jnp
from jax import lax
from jax.experimental import pallas as pl
from jax.experimental.pallas import tpu as pltpu
from jax.experimental.pallas import tpu_sc as plsc

N_NODES = 10000
N_EDGES = 320000
D_FEAT = 128
D_EDGE = 16
D_OUT = 16
N_PK = N_EDGES // 8


def _nodeproj_body(x_ref, w_ref, xs_ref, xd_ref):
    xs_ref[...] = jnp.dot(x_ref[...], w_ref[D_EDGE:D_EDGE + D_FEAT, :],
                          preferred_element_type=jnp.float32)
    xd_ref[...] = jnp.dot(x_ref[...], w_ref[D_EDGE + D_FEAT:, :],
                          preferred_element_type=jnp.float32)


def _nodeproj(x, W):
    return pl.pallas_call(
        _nodeproj_body,
        out_shape=[
            jax.ShapeDtypeStruct((N_NODES, D_OUT), jnp.float32),
            jax.ShapeDtypeStruct((N_NODES, D_OUT), jnp.float32),
        ],
    )(x, W)


_G = 2000
_EBLK = 8 * _G
_NBLK = N_EDGES // _EBLK

_CONTRACT0 = (((0,), (0,)), ((), ()))
_CONTRACT1 = (((1,), (1,)), ((), ()))


def _finish_body(gs_ref, eat_ref, w_ref, b_ref, eye_ref, out_ref):
    p = gs_ref[...]
    eye = eye_ref[...]
    parts = []
    for j in range(8):
        sl = p[:, j * D_OUT:(j + 1) * D_OUT]
        parts.append(lax.dot_general(eye, sl, _CONTRACT1,
                                     preferred_element_type=jnp.float32))
    gsT = jnp.concatenate(parts, axis=1)
    eapT = lax.dot_general(w_ref[:D_EDGE, :], eat_ref[...], _CONTRACT0,
                           preferred_element_type=jnp.float32)
    out_ref[...] = gsT + eapT + b_ref[...]


def _finish(gsum_pk, eaT, W, bcol, eye16):
    return pl.pallas_call(
        _finish_body,
        grid=(_NBLK,),
        in_specs=[
            pl.BlockSpec((_G, 128), lambda i: (i, 0)),
            pl.BlockSpec((D_EDGE, _EBLK), lambda i: (0, i)),
            pl.BlockSpec((D_EDGE + 2 * D_FEAT, D_OUT), lambda i: (0, 0)),
            pl.BlockSpec((D_OUT, 1), lambda i: (0, 0)),
            pl.BlockSpec((D_OUT, D_OUT), lambda i: (0, 0)),
        ],
        out_specs=pl.BlockSpec((D_OUT, _EBLK), lambda i: (0, i)),
        out_shape=jax.ShapeDtypeStruct((D_OUT, N_EDGES), jnp.float32),
    )(gsum_pk, eaT, W, bcol, eye16)


_NW = 32
_EPW = N_EDGES // _NW
_MACRO = 2000
_PMACRO = _MACRO // 8
_NMACRO = _EPW // _MACRO

_mesh = plsc.VectorSubcoreMesh(core_axis_name="c", subcore_axis_name="s")


@functools.partial(
    pl.kernel,
    mesh=_mesh,
    compiler_params=pltpu.CompilerParams(use_tc_tiling_on_sc=False),
    out_type=jax.ShapeDtypeStruct((N_PK, 128), jnp.float32),
    scratch_types=[
        pltpu.VMEM((_EPW,), jnp.int32),
        pltpu.VMEM((_EPW,), jnp.int32),
        pltpu.VMEM((_MACRO, D_OUT), jnp.float32),
        pltpu.VMEM((_MACRO, D_OUT), jnp.float32),
        pltpu.SemaphoreType.DMA,
        pltpu.SemaphoreType.DMA,
        pltpu.SemaphoreType.DMA,
        pltpu.SemaphoreType.DMA,
        pltpu.SemaphoreType.DMA,
        pltpu.SemaphoreType.DMA,
    ],
)
def _sc_gather_sum(src_hbm, dst_hbm, xs_hbm, xd_hbm, out_hbm,
                   idx_s, idx_d, res0, res1,
                   sem_g0, sem_g1, sem_a0, sem_a1, sem_w0, sem_w1):
    wid = lax.axis_index("s") * 2 + lax.axis_index("c")
    base = wid * _EPW
    pltpu.sync_copy(src_hbm.at[pl.ds(base, _EPW)], idx_s)
    pltpu.sync_copy(dst_hbm.at[pl.ds(base, _EPW)], idx_d)
    res = [res0, res1]
    sem_g = [sem_g0, sem_g1]
    sem_a = [sem_a0, sem_a1]
    sem_w = [sem_w0, sem_w1]

    def dst_slice(m):
        gm = (base + m * _MACRO) // _MACRO
        prow = (gm // 8) * _G
        j = gm % 8
        return out_hbm.at[pl.ds(prow, _G), pl.ds(j * D_OUT, D_OUT)]

    P = [None] * _NMACRO
    A = [None] * _NMACRO
    Wr = [None] * _NMACRO
    P[0] = pltpu.async_copy(xs_hbm.at[idx_s.at[pl.ds(0, _MACRO)]],
                            res[0], sem_g[0])
    for m in range(_NMACRO):
        b = m % 2
        P[m].wait()
        A[m] = pltpu.async_copy(xd_hbm.at[idx_d.at[pl.ds(m * _MACRO, _MACRO)]],
                                res[b], sem_a[b], add=True)
        A[m].wait()
        if m + 1 < _NMACRO:
            if m >= 1:
                Wr[m - 1].wait()
            P[m + 1] = pltpu.async_copy(
                xs_hbm.at[idx_s.at[pl.ds((m + 1) * _MACRO, _MACRO)]],
                res[1 - b], sem_g[1 - b])
        Wr[m] = pltpu.async_copy(res[b], dst_slice(m), sem_w[b])
    Wr[_NMACRO - 2].wait()
    Wr[_NMACRO - 1].wait()


def kernel(x, edge_index, edge_attr, pos, W, b):
    xs, xd = _nodeproj(x, W)
    gsum_pk = _sc_gather_sum(edge_index[0], edge_index[1], xs, xd)
    outT = _finish(gsum_pk, edge_attr.T, W, b.reshape(D_OUT, 1),
                   jnp.eye(D_OUT, dtype=jnp.float32))
    return (x, outT.T, edge_index, pos)
